# baseline (speedup 1.0000x reference)
import jax
import jax.numpy as jnp
from jax import lax
from jax.experimental import pallas as pl
from jax.experimental.pallas import tpu as pltpu
from jax.experimental.pallas import tpu_sc as plsc

_B = 16384
_D = 64
_UN = 1000000
_NC = 2
_NS = 16
_NW = _NC * _NS
_RPW = _B // _NW
_L = 16
_HALF = _RPW // 2
_HW = _HALF * _D
_CPAD = 17


def _body(user_hbm, item_hbm, tlin_hbm, a_hbm, b_hbm, out_hbm,
          idx_u, idx_i, idx_t, th_v, a_v, b_v, c_v, out_v, sem, semb):
    wid = lax.axis_index("s") * _NC + lax.axis_index("c")
    base = wid * _RPW

    pltpu.sync_copy(user_hbm.at[pl.ds(base, _RPW)], idx_u)
    pltpu.sync_copy(item_hbm.at[pl.ds(base, _RPW)], idx_i)

    bcps = [pltpu.async_copy(b_hbm.at[idx_i.at[pl.ds(j * 128, 128)]],
                             b_v.at[pl.ds(j * 128, 128)], semb)
            for j in range(_RPW // 128)]

    dvecs = [(jnp.arange(_L, dtype=jnp.int32) + k * _L) * _UN
             for k in range(_D // _L)]

    for half in range(2):
        hbase = half * _HALF

        @pl.loop(0, _HALF // _L)
        def _bld(c):
            iu = idx_u[pl.ds(hbase + c * _L, _L)]
            for j in range(_L):
                e = c * _L + j
                for k in range(_D // _L):
                    idx_t[pl.ds(e * _D + k * _L, _L)] = dvecs[k] + iu[j]

        tcps = [pltpu.async_copy(tlin_hbm.at[idx_t.at[pl.ds(j * 128, 128)]],
                                 th_v.at[pl.ds(j * 128, 128)], sem)
                for j in range(_HW // 128)]

        @pl.loop(0, _HALF // _L)
        def _chunk(c):
            ii = idx_i[pl.ds(hbase + c * _L, _L)]
            cps = [pltpu.async_copy(a_hbm.at[ii[j]], a_v.at[c * _L + j], semb)
                   for j in range(_L)]
            for cp in cps:
                cp.wait()

        for cp in tcps:
            cp.wait()
        if half == 0:
            for cp in bcps:
                cp.wait()

        @pl.loop(0, _HALF)
        def _row(i):
            acc = jnp.zeros((_L,), jnp.float32)
            for k in range(_D // _L):
                th = th_v[pl.ds(i * _D + k * _L, _L)]
                ar = a_v[i, pl.ds(k * _L, _L)]
                acc = acc + th / (1.0 + jnp.exp(-ar))
            c_v[pl.ds(i * _CPAD, _L)] = acc

        lane = lax.iota(jnp.int32, _L)
        @pl.loop(0, _HALF // _L)
        def _grp(g):
            rowbase = g * (_L * _CPAD)
            dot = jnp.zeros((_L,), jnp.float32)
            for d in range(_L):
                dot = dot + plsc.load_gather(c_v, [rowbase + lane * _CPAD + d])
            bv = b_v[pl.ds(hbase + g * _L, _L)]
            out_v[pl.ds(hbase + g * _L, _L)] = 1.0 / (1.0 + jnp.exp(bv - dot))

    pltpu.sync_copy(out_v, out_hbm.at[pl.ds(base, _RPW)])


def kernel(user, item, theta_table, a_table, b_table):
    user = user.astype(jnp.int32)
    item = item.astype(jnp.int32)
    t_lin = theta_table.T.reshape(-1)
    b_lin = b_table.T.reshape(-1)
    mesh = plsc.VectorSubcoreMesh(
        core_axis_name="c", subcore_axis_name="s",
        num_cores=_NC, num_subcores=_NS)
    ker = pl.kernel(
        _body,
        out_type=jax.ShapeDtypeStruct((_B,), jnp.float32),
        mesh=mesh,
        compiler_params=pltpu.CompilerParams(needs_layout_passes=False),
        scratch_types=[
            pltpu.VMEM((_RPW,), jnp.int32),
            pltpu.VMEM((_RPW,), jnp.int32),
            pltpu.VMEM((_HW,), jnp.int32),
            pltpu.VMEM((_HW,), jnp.float32),
            pltpu.VMEM((_HALF, _D), jnp.float32),
            pltpu.VMEM((_RPW,), jnp.float32),
            pltpu.VMEM((_HALF * _CPAD,), jnp.float32),
            pltpu.VMEM((_RPW,), jnp.float32),
            pltpu.SemaphoreType.DMA,
            pltpu.SemaphoreType.DMA,
        ],
    )
    return ker(user, item, t_lin, a_table, b_lin)


# speedup vs baseline: 11.9913x; 11.9913x over previous
import jax
import jax.numpy as jnp
from jax import lax
from jax.experimental import pallas as pl
from jax.experimental.pallas import tpu as pltpu
from jax.experimental.pallas import tpu_sc as plsc

_B = 16384
_D = 64
_NC = 2
_NS = 16
_NW = _NC * _NS
_RPW = _B // _NW
_L = 16
_HALF = _RPW // 2
_CPAD = 17


def _body(user_hbm, item_hbm, theta_hbm, a_hbm, b_hbm, out_hbm,
          idx_u, idx_i, th_v, a_v, b_v, c_v, out_v, sem, semb):
    wid = lax.axis_index("s") * _NC + lax.axis_index("c")
    base = wid * _RPW

    pltpu.sync_copy(user_hbm.at[pl.ds(base, _RPW)], idx_u)
    pltpu.sync_copy(item_hbm.at[pl.ds(base, _RPW)], idx_i)

    bcps = [pltpu.async_copy(b_hbm.at[idx_i.at[pl.ds(j * 128, 128)]],
                             b_v.at[pl.ds(j * 128, 128)], semb)
            for j in range(_RPW // 128)]

    for half in range(2):
        hbase = half * _HALF

        @pl.loop(0, _HALF // _L)
        def _chunk(c):
            iu = idx_u[pl.ds(hbase + c * _L, _L)]
            ii = idx_i[pl.ds(hbase + c * _L, _L)]
            cps = []
            for j in range(_L):
                r = c * _L + j
                cps.append(pltpu.async_copy(theta_hbm.at[iu[j]], th_v.at[r], sem))
                cps.append(pltpu.async_copy(a_hbm.at[ii[j]], a_v.at[r], sem))
            for cp in cps:
                cp.wait()

        @pl.loop(0, _HALF)
        def _row(i):
            acc = jnp.zeros((_L,), jnp.float32)
            for k in range(_D // _L):
                th = th_v[i, pl.ds(k * _L, _L)]
                ar = a_v[i, pl.ds(k * _L, _L)]
                acc = acc + th / (1.0 + jnp.exp(-ar))
            c_v[pl.ds(i * _CPAD, _L)] = acc

        if half == 0:
            for cp in bcps:
                cp.wait()

        lane = lax.iota(jnp.int32, _L)
        @pl.loop(0, _HALF // _L)
        def _grp(g):
            rowbase = g * (_L * _CPAD)
            dot = jnp.zeros((_L,), jnp.float32)
            for d in range(_L):
                dot = dot + plsc.load_gather(c_v, [rowbase + lane * _CPAD + d])
            bv = b_v[pl.ds(hbase + g * _L, _L)]
            out_v[pl.ds(hbase + g * _L, _L)] = 1.0 / (1.0 + jnp.exp(bv - dot))

    pltpu.sync_copy(out_v, out_hbm.at[pl.ds(base, _RPW)])


def kernel(user, item, theta_table, a_table, b_table):
    user = user.astype(jnp.int32)
    item = item.astype(jnp.int32)
    b_lin = b_table.reshape(-1)
    mesh = plsc.VectorSubcoreMesh(
        core_axis_name="c", subcore_axis_name="s",
        num_cores=_NC, num_subcores=_NS)
    ker = pl.kernel(
        _body,
        out_type=jax.ShapeDtypeStruct((_B,), jnp.float32),
        mesh=mesh,
        compiler_params=pltpu.CompilerParams(needs_layout_passes=False),
        scratch_types=[
            pltpu.VMEM((_RPW,), jnp.int32),
            pltpu.VMEM((_RPW,), jnp.int32),
            pltpu.VMEM((_HALF, _D), jnp.float32),
            pltpu.VMEM((_HALF, _D), jnp.float32),
            pltpu.VMEM((_RPW,), jnp.float32),
            pltpu.VMEM((_HALF * _CPAD,), jnp.float32),
            pltpu.VMEM((_RPW,), jnp.float32),
            pltpu.SemaphoreType.DMA,
            pltpu.SemaphoreType.DMA,
        ],
    )
    return ker(user, item, theta_table, a_table, b_lin)
